# baseline (speedup 1.0000x reference)
import functools

import jax
import jax.numpy as jnp
import numpy as np
from jax import lax
from jax.experimental import pallas as pl
from jax.experimental.pallas import tpu as pltpu
from jax.experimental.pallas import tpu_sc as plsc

NUM_EXPERTS = 8
TOP_K = 2
D_MODEL = 1024
D_FF = 4096
N_TOKENS = 4096

_BT = 128
_NB = (N_TOKENS * TOP_K + NUM_EXPERTS * (_BT - 1) + _BT - 1) // _BT
_DISP = _NB * _BT
_NA = N_TOKENS * TOP_K

_EPS = np.float32(np.finfo(float).eps)
_PREC = jax.lax.Precision.DEFAULT

_NC = 2
_NS = 16
_NW = _NC * _NS


def _dot(a, b):
    return jax.lax.dot_general(a, b, (((1,), (0,)), ((), ())),
                               preferred_element_type=jnp.float32,
                               precision=_PREC)


_CH = 512
_NCH = N_TOKENS // _CH


def _router_body(x_ref, wg_ref, g01_ref, dest_ref, bexp_ref, nused_ref,
                 loss_ref, m_ref, oh1_ref):
    x = x_ref[...]
    logits = _dot(x, wg_ref[...])
    e_ids = jax.lax.broadcasted_iota(jnp.int32, logits.shape, 1)

    m1 = jnp.max(logits, axis=1, keepdims=True)
    idx1 = jnp.min(jnp.where(logits == m1, e_ids, NUM_EXPERTS), axis=1,
                   keepdims=True)
    masked = jnp.where(e_ids == idx1, -jnp.inf, logits)
    m2 = jnp.max(masked, axis=1, keepdims=True)
    idx2 = jnp.min(jnp.where(masked == m2, e_ids, NUM_EXPERTS), axis=1,
                   keepdims=True)

    ex = jnp.exp(m2 - m1)
    denom = 1.0 + ex
    g1 = 1.0 / denom
    g2 = ex / denom
    g01_ref[...] = jnp.concatenate([g1, g2], axis=1)

    oh1 = (e_ids == idx1).astype(jnp.float32)
    oh2 = (e_ids == idx2).astype(jnp.float32)
    sel = oh1 + oh2
    m_ref[...] = sel
    oh1_ref[...] = oh1

    gates = oh1 * g1 + oh2 * g2
    importance = jnp.sum(gates, axis=0)
    load = jnp.sum((gates > 0.0).astype(jnp.float32), axis=0)

    def cv_sq(v):
        mean = jnp.mean(v)
        var = jnp.sum((v - mean) ** 2) / np.float32(NUM_EXPERTS - 1)
        return var / (mean * mean + np.float32(1e-10))

    loss = (cv_sq(importance) + cv_sq(load)) * np.float32(0.01)
    loss_ref[...] = jnp.reshape(loss, (1, 1))

    counts = jnp.sum(sel, axis=0, keepdims=True)
    nblk = jnp.ceil(counts / np.float32(_BT))
    r8 = jax.lax.broadcasted_iota(jnp.int32, (NUM_EXPERTS, NUM_EXPERTS), 0)
    c8 = jax.lax.broadcasted_iota(jnp.int32, (NUM_EXPERTS, NUM_EXPERTS), 1)
    upper8 = (r8 < c8).astype(jnp.float32)
    blk_off = _dot(nblk, upper8)
    off_rows = blk_off * np.float32(_BT)
    ends = jnp.reshape(blk_off + nblk, (NUM_EXPERTS, 1))
    used = jnp.sum(nblk)

    bb = jax.lax.broadcasted_iota(jnp.int32, (NUM_EXPERTS, _NB), 1)
    raw = jnp.sum((bb.astype(jnp.float32) >= ends).astype(jnp.float32),
                  axis=0, keepdims=True)
    eid8 = jax.lax.broadcasted_iota(jnp.int32, (1, NUM_EXPERTS), 1)
    laste = jnp.max(jnp.where(counts > 0.0, eid8, -1)).astype(jnp.float32)
    bexp_ref[...] = jnp.minimum(raw, laste).astype(jnp.int32)
    nused_ref[...] = jnp.reshape(used, (1, 1)).astype(jnp.int32)

    rr = jax.lax.broadcasted_iota(jnp.int32, (_CH, _CH), 0)
    cc = jax.lax.broadcasted_iota(jnp.int32, (_CH, _CH), 1)
    lower = (rr > cc).astype(jnp.float32)

    def chunk(c, run):
        mc = m_ref[pl.ds(c * _CH, _CH), :]
        oh1c = oh1_ref[pl.ds(c * _CH, _CH), :]
        oh2c = mc - oh1c
        pos = _dot(lower, mc) + run
        slot = pos + off_rows
        d0 = jnp.sum(oh1c * slot, axis=1, keepdims=True)
        d1 = jnp.sum(oh2c * slot, axis=1, keepdims=True)
        dest_ref[pl.ds(c * _CH, _CH), :] = jnp.concatenate(
            [d0, d1], axis=1).astype(jnp.int32)
        return run + jnp.sum(mc, axis=0, keepdims=True)

    jax.lax.fori_loop(0, _NCH, chunk, jnp.zeros((1, NUM_EXPERTS),
                                                jnp.float32))


def _router(x, w_gate):
    return pl.pallas_call(
        _router_body,
        out_shape=(
            jax.ShapeDtypeStruct((N_TOKENS, 2), jnp.float32),
            jax.ShapeDtypeStruct((N_TOKENS, 2), jnp.int32),
            jax.ShapeDtypeStruct((1, _NB), jnp.int32),
            jax.ShapeDtypeStruct((1, 1), jnp.int32),
            jax.ShapeDtypeStruct((1, 1), jnp.float32),
        ),
        scratch_shapes=[
            pltpu.VMEM((N_TOKENS, NUM_EXPERTS), jnp.float32),
            pltpu.VMEM((N_TOKENS, NUM_EXPERTS), jnp.float32),
        ],
    )(x, w_gate)


_T_PER_W = N_TOKENS // _NW
_SC_CH = 16


def _sc_scatter_body(x_hbm, d0_hbm, d1_hbm, xd_hbm, dv0, dv1, xv, sem):
    wid = lax.axis_index("s") * _NC + lax.axis_index("c")
    base_t = wid * _T_PER_W
    pltpu.sync_copy(d0_hbm.at[wid], dv0)
    pltpu.sync_copy(d1_hbm.at[wid], dv1)
    for j in range(_T_PER_W // _SC_CH):
        pltpu.sync_copy(x_hbm.at[pl.ds(base_t + j * _SC_CH, _SC_CH)], xv)
        idx0 = dv0[j]
        idx1 = dv1[j]
        cp0 = pltpu.async_copy(xv, xd_hbm.at[idx0], sem)
        cp1 = pltpu.async_copy(xv, xd_hbm.at[idx1], sem)
        cp0.wait()
        cp1.wait()


def _sc_scatter(x, dest0_3d, dest1_3d):
    mesh = plsc.VectorSubcoreMesh(core_axis_name="c", subcore_axis_name="s")
    nj = _T_PER_W // _SC_CH
    return pl.kernel(
        _sc_scatter_body,
        out_type=jax.ShapeDtypeStruct((_DISP, D_MODEL), jnp.float32),
        mesh=mesh,
        scratch_types=[
            pltpu.VMEM((nj, _SC_CH), jnp.int32),
            pltpu.VMEM((nj, _SC_CH), jnp.int32),
            pltpu.VMEM((_SC_CH, D_MODEL), jnp.float32),
            pltpu.SemaphoreType.DMA,
        ],
    )(x, dest0_3d, dest1_3d)


def _mlp1_body(bexp_ref, nu_ref, xd_ref, w1_ref, b1_ref, h_ref):
    b = pl.program_id(0)

    @pl.when(b < nu_ref[0])
    def _():
        xb = xd_ref[...].astype(jnp.bfloat16)
        w1b = w1_ref[0].astype(jnp.bfloat16)
        h = jnp.maximum(_dot(xb, w1b) + b1_ref[0], 0.0)
        h_ref[...] = h.astype(jnp.bfloat16)


def _mlp1(xd, w1, b1, bexp, nused):
    grid_spec = pltpu.PrefetchScalarGridSpec(
        num_scalar_prefetch=2,
        grid=(_NB,),
        in_specs=[
            pl.BlockSpec((_BT, D_MODEL),
                         lambda b, be, nu: (jnp.minimum(b, nu[0] - 1), 0)),
            pl.BlockSpec((1, D_MODEL, D_FF), lambda b, be, nu: (be[b], 0, 0)),
            pl.BlockSpec((1, 1, D_FF), lambda b, be, nu: (be[b], 0, 0)),
        ],
        out_specs=pl.BlockSpec((_BT, D_FF), lambda b, be, nu: (b, 0)),
    )
    return pl.pallas_call(
        _mlp1_body,
        grid_spec=grid_spec,
        out_shape=jax.ShapeDtypeStruct((_DISP, D_FF), jnp.bfloat16),
        compiler_params=pltpu.CompilerParams(
            dimension_semantics=("arbitrary",),
        ),
    )(bexp, nused, xd, w1, b1[:, None, :])


def _mlp2_body(bexp_ref, nu_ref, h_ref, w2_ref, b2_ref, eo_ref):
    b = pl.program_id(0)

    @pl.when(b < nu_ref[0])
    def _():
        w2b = w2_ref[0].astype(jnp.bfloat16)
        eo_ref[...] = jnp.exp(_dot(h_ref[...], w2b) + b2_ref[0])


def _mlp2(h, w2, b2, bexp, nused):
    grid_spec = pltpu.PrefetchScalarGridSpec(
        num_scalar_prefetch=2,
        grid=(_NB,),
        in_specs=[
            pl.BlockSpec((_BT, D_FF),
                         lambda b, be, nu: (jnp.minimum(b, nu[0] - 1), 0)),
            pl.BlockSpec((1, D_FF, D_MODEL), lambda b, be, nu: (be[b], 0, 0)),
            pl.BlockSpec((1, 1, D_MODEL), lambda b, be, nu: (be[b], 0, 0)),
        ],
        out_specs=pl.BlockSpec((_BT, D_MODEL), lambda b, be, nu: (b, 0)),
    )
    return pl.pallas_call(
        _mlp2_body,
        grid_spec=grid_spec,
        out_shape=jax.ShapeDtypeStruct((_DISP, D_MODEL), jnp.float32),
        compiler_params=pltpu.CompilerParams(
            dimension_semantics=("arbitrary",),
        ),
    )(bexp, nused, h, w2, b2[:, None, :])


_A_PER_W = _NA // _NW
_G_CH = 32


def _sc_gather_body(eo_hbm, d0_hbm, d1_hbm, out_hbm, dv0, dv1, rows0, rows1,
                    sem):
    wid = lax.axis_index("s") * _NC + lax.axis_index("c")
    base_t = wid * _T_PER_W
    pltpu.sync_copy(d0_hbm.at[wid], dv0)
    pltpu.sync_copy(d1_hbm.at[wid], dv1)
    for j in range(_T_PER_W // _G_CH):
        cp0 = pltpu.async_copy(eo_hbm.at[dv0.at[j]], rows0, sem)
        cp1 = pltpu.async_copy(eo_hbm.at[dv1.at[j]], rows1, sem)
        cp0.wait()
        cp1.wait()
        tb = base_t + j * _G_CH
        pltpu.sync_copy(rows0, out_hbm.at[pl.ds(tb, _G_CH), 0, :])
        pltpu.sync_copy(rows1, out_hbm.at[pl.ds(tb, _G_CH), 1, :])


def _sc_gather(eo, dest0_3dg, dest1_3dg):
    mesh = plsc.VectorSubcoreMesh(core_axis_name="c", subcore_axis_name="s")
    nj = _T_PER_W // _G_CH
    return pl.kernel(
        _sc_gather_body,
        out_type=jax.ShapeDtypeStruct((N_TOKENS, 2, D_MODEL), jnp.float32),
        mesh=mesh,
        scratch_types=[
            pltpu.VMEM((nj, _G_CH), jnp.int32),
            pltpu.VMEM((nj, _G_CH), jnp.int32),
            pltpu.VMEM((_G_CH, D_MODEL), jnp.float32),
            pltpu.VMEM((_G_CH, D_MODEL), jnp.float32),
            pltpu.SemaphoreType.DMA,
        ],
    )(eo, dest0_3dg, dest1_3dg)


_BC = 512


def _combine_body(buf_ref, g_ref, y_ref):
    g = g_ref[...]
    cols = jax.lax.broadcasted_iota(jnp.int32, g.shape, 1)
    g0 = jnp.sum(jnp.where(cols == 0, g, 0.0), axis=1, keepdims=True)
    g1 = jnp.sum(jnp.where(cols == 1, g, 0.0), axis=1, keepdims=True)
    s = g0 * buf_ref[:, 0, :] + g1 * buf_ref[:, 1, :]
    y_ref[...] = jnp.log(jnp.where(s == 0.0, _EPS, s))


def _combine(buf, g01):
    return pl.pallas_call(
        _combine_body,
        grid=(N_TOKENS // _BC,),
        in_specs=[
            pl.BlockSpec((_BC, 2, D_MODEL), lambda t: (t, 0, 0)),
            pl.BlockSpec((_BC, 2), lambda t: (t, 0)),
        ],
        out_specs=pl.BlockSpec((_BC, D_MODEL), lambda t: (t, 0)),
        out_shape=jax.ShapeDtypeStruct((N_TOKENS, D_MODEL), jnp.float32),
    )(buf, g01)


def kernel(x, w_gate, w1, b1, w2, b2):
    g01, dest, bexp, nused, loss = _router(x, w_gate)
    nj = _T_PER_W // _SC_CH
    dest0_3d = jnp.reshape(dest[:, 0], (_NW, nj, _SC_CH))
    dest1_3d = jnp.reshape(dest[:, 1], (_NW, nj, _SC_CH))
    xd = _sc_scatter(x, dest0_3d, dest1_3d)
    bexp_v = jnp.reshape(bexp, (_NB,))
    nused_v = jnp.reshape(nused, (1,))
    h = _mlp1(xd, w1, b1, bexp_v, nused_v)
    eo = _mlp2(h, w2, b2, bexp_v, nused_v)
    njg = _T_PER_W // _G_CH
    dest0_3dg = jnp.reshape(dest[:, 0], (_NW, njg, _G_CH))
    dest1_3dg = jnp.reshape(dest[:, 1], (_NW, njg, _G_CH))
    buf = _sc_gather(eo, dest0_3dg, dest1_3dg)
    y = _combine(buf, g01)
    return (y, loss[0, 0])


# speedup vs baseline: 1.0623x; 1.0623x over previous
import functools

import jax
import jax.numpy as jnp
import numpy as np
from jax import lax
from jax.experimental import pallas as pl
from jax.experimental.pallas import tpu as pltpu
from jax.experimental.pallas import tpu_sc as plsc

NUM_EXPERTS = 8
TOP_K = 2
D_MODEL = 1024
D_FF = 4096
N_TOKENS = 4096

_BT = 256
_NB = (N_TOKENS * TOP_K + NUM_EXPERTS * (_BT - 1) + _BT - 1) // _BT
_DISP = _NB * _BT
_NA = N_TOKENS * TOP_K

_EPS = np.float32(np.finfo(float).eps)
_PREC = jax.lax.Precision.DEFAULT

_NC = 2
_NS = 16
_NW = _NC * _NS


def _dot(a, b):
    return jax.lax.dot_general(a, b, (((1,), (0,)), ((), ())),
                               preferred_element_type=jnp.float32,
                               precision=_PREC)


_CH = 512
_NCH = N_TOKENS // _CH


def _router_body(x_ref, wg_ref, g01_ref, dest_ref, bexp_ref, nused_ref,
                 loss_ref, m_ref, oh1_ref):
    x = x_ref[...]
    logits = _dot(x, wg_ref[...])
    e_ids = jax.lax.broadcasted_iota(jnp.int32, logits.shape, 1)

    m1 = jnp.max(logits, axis=1, keepdims=True)
    idx1 = jnp.min(jnp.where(logits == m1, e_ids, NUM_EXPERTS), axis=1,
                   keepdims=True)
    masked = jnp.where(e_ids == idx1, -jnp.inf, logits)
    m2 = jnp.max(masked, axis=1, keepdims=True)
    idx2 = jnp.min(jnp.where(masked == m2, e_ids, NUM_EXPERTS), axis=1,
                   keepdims=True)

    ex = jnp.exp(m2 - m1)
    denom = 1.0 + ex
    g1 = 1.0 / denom
    g2 = ex / denom
    g01_ref[...] = jnp.concatenate([g1, g2], axis=1)

    oh1 = (e_ids == idx1).astype(jnp.float32)
    oh2 = (e_ids == idx2).astype(jnp.float32)
    sel = oh1 + oh2
    m_ref[...] = sel
    oh1_ref[...] = oh1

    gates = oh1 * g1 + oh2 * g2
    importance = jnp.sum(gates, axis=0)
    load = jnp.sum((gates > 0.0).astype(jnp.float32), axis=0)

    def cv_sq(v):
        mean = jnp.mean(v)
        var = jnp.sum((v - mean) ** 2) / np.float32(NUM_EXPERTS - 1)
        return var / (mean * mean + np.float32(1e-10))

    loss = (cv_sq(importance) + cv_sq(load)) * np.float32(0.01)
    loss_ref[...] = jnp.reshape(loss, (1, 1))

    counts = jnp.sum(sel, axis=0, keepdims=True)
    nblk = jnp.ceil(counts / np.float32(_BT))
    r8 = jax.lax.broadcasted_iota(jnp.int32, (NUM_EXPERTS, NUM_EXPERTS), 0)
    c8 = jax.lax.broadcasted_iota(jnp.int32, (NUM_EXPERTS, NUM_EXPERTS), 1)
    upper8 = (r8 < c8).astype(jnp.float32)
    blk_off = _dot(nblk, upper8)
    off_rows = blk_off * np.float32(_BT)
    ends = jnp.reshape(blk_off + nblk, (NUM_EXPERTS, 1))
    used = jnp.sum(nblk)

    bb = jax.lax.broadcasted_iota(jnp.int32, (NUM_EXPERTS, _NB), 1)
    raw = jnp.sum((bb.astype(jnp.float32) >= ends).astype(jnp.float32),
                  axis=0, keepdims=True)
    eid8 = jax.lax.broadcasted_iota(jnp.int32, (1, NUM_EXPERTS), 1)
    laste = jnp.max(jnp.where(counts > 0.0, eid8, -1)).astype(jnp.float32)
    bexp_ref[...] = jnp.minimum(raw, laste).astype(jnp.int32)
    nused_ref[...] = jnp.reshape(used, (1, 1)).astype(jnp.int32)

    rr = jax.lax.broadcasted_iota(jnp.int32, (_CH, _CH), 0)
    cc = jax.lax.broadcasted_iota(jnp.int32, (_CH, _CH), 1)
    lower = (rr > cc).astype(jnp.float32)

    def chunk(c, run):
        mc = m_ref[pl.ds(c * _CH, _CH), :]
        oh1c = oh1_ref[pl.ds(c * _CH, _CH), :]
        oh2c = mc - oh1c
        pos = _dot(lower, mc) + run
        slot = pos + off_rows
        d0 = jnp.sum(oh1c * slot, axis=1, keepdims=True)
        d1 = jnp.sum(oh2c * slot, axis=1, keepdims=True)
        dest_ref[pl.ds(c * _CH, _CH), :] = jnp.concatenate(
            [d0, d1], axis=1).astype(jnp.int32)
        return run + jnp.sum(mc, axis=0, keepdims=True)

    jax.lax.fori_loop(0, _NCH, chunk, jnp.zeros((1, NUM_EXPERTS),
                                                jnp.float32))


def _router(x, w_gate):
    return pl.pallas_call(
        _router_body,
        out_shape=(
            jax.ShapeDtypeStruct((N_TOKENS, 2), jnp.float32),
            jax.ShapeDtypeStruct((N_TOKENS, 2), jnp.int32),
            jax.ShapeDtypeStruct((1, _NB), jnp.int32),
            jax.ShapeDtypeStruct((1, 1), jnp.int32),
            jax.ShapeDtypeStruct((1, 1), jnp.float32),
        ),
        scratch_shapes=[
            pltpu.VMEM((N_TOKENS, NUM_EXPERTS), jnp.float32),
            pltpu.VMEM((N_TOKENS, NUM_EXPERTS), jnp.float32),
        ],
    )(x, w_gate)


_T_PER_W = N_TOKENS // _NW
_SC_CH = 16


def _sc_scatter_body(x_hbm, d0_hbm, d1_hbm, xd_hbm, dv0, dv1, xv, sem):
    wid = lax.axis_index("s") * _NC + lax.axis_index("c")
    base_t = wid * _T_PER_W
    pltpu.sync_copy(d0_hbm.at[wid], dv0)
    pltpu.sync_copy(d1_hbm.at[wid], dv1)
    for j in range(_T_PER_W // _SC_CH):
        pltpu.sync_copy(x_hbm.at[pl.ds(base_t + j * _SC_CH, _SC_CH)], xv)
        idx0 = dv0[j]
        idx1 = dv1[j]
        cp0 = pltpu.async_copy(xv, xd_hbm.at[idx0], sem)
        cp1 = pltpu.async_copy(xv, xd_hbm.at[idx1], sem)
        cp0.wait()
        cp1.wait()


def _sc_scatter(x, dest0_3d, dest1_3d):
    mesh = plsc.VectorSubcoreMesh(core_axis_name="c", subcore_axis_name="s")
    nj = _T_PER_W // _SC_CH
    return pl.kernel(
        _sc_scatter_body,
        out_type=jax.ShapeDtypeStruct((_DISP, D_MODEL), jnp.float32),
        mesh=mesh,
        scratch_types=[
            pltpu.VMEM((nj, _SC_CH), jnp.int32),
            pltpu.VMEM((nj, _SC_CH), jnp.int32),
            pltpu.VMEM((_SC_CH, D_MODEL), jnp.float32),
            pltpu.SemaphoreType.DMA,
        ],
    )(x, dest0_3d, dest1_3d)


def _mlp1_body(bexp_ref, nu_ref, xd_ref, w1_ref, b1_ref, h_ref):
    b = pl.program_id(0)

    @pl.when(b < nu_ref[0])
    def _():
        xb = xd_ref[...].astype(jnp.bfloat16)
        w1b = w1_ref[0].astype(jnp.bfloat16)
        h = jnp.maximum(_dot(xb, w1b) + b1_ref[0], 0.0)
        h_ref[...] = h.astype(jnp.bfloat16)


def _mlp1(xd, w1, b1, bexp, nused):
    grid_spec = pltpu.PrefetchScalarGridSpec(
        num_scalar_prefetch=2,
        grid=(_NB,),
        in_specs=[
            pl.BlockSpec((_BT, D_MODEL),
                         lambda b, be, nu: (jnp.minimum(b, nu[0] - 1), 0)),
            pl.BlockSpec((1, D_MODEL, D_FF), lambda b, be, nu: (be[b], 0, 0)),
            pl.BlockSpec((1, 1, D_FF), lambda b, be, nu: (be[b], 0, 0)),
        ],
        out_specs=pl.BlockSpec((_BT, D_FF), lambda b, be, nu: (b, 0)),
    )
    return pl.pallas_call(
        _mlp1_body,
        grid_spec=grid_spec,
        out_shape=jax.ShapeDtypeStruct((_DISP, D_FF), jnp.bfloat16),
        compiler_params=pltpu.CompilerParams(
            dimension_semantics=("arbitrary",),
        ),
    )(bexp, nused, xd, w1, b1[:, None, :])


def _mlp2_body(bexp_ref, nu_ref, h_ref, w2_ref, b2_ref, eo_ref):
    b = pl.program_id(0)

    @pl.when(b < nu_ref[0])
    def _():
        w2b = w2_ref[0].astype(jnp.bfloat16)
        eo_ref[...] = jnp.exp(_dot(h_ref[...], w2b) + b2_ref[0])


def _mlp2(h, w2, b2, bexp, nused):
    grid_spec = pltpu.PrefetchScalarGridSpec(
        num_scalar_prefetch=2,
        grid=(_NB,),
        in_specs=[
            pl.BlockSpec((_BT, D_FF),
                         lambda b, be, nu: (jnp.minimum(b, nu[0] - 1), 0)),
            pl.BlockSpec((1, D_FF, D_MODEL), lambda b, be, nu: (be[b], 0, 0)),
            pl.BlockSpec((1, 1, D_MODEL), lambda b, be, nu: (be[b], 0, 0)),
        ],
        out_specs=pl.BlockSpec((_BT, D_MODEL), lambda b, be, nu: (b, 0)),
    )
    return pl.pallas_call(
        _mlp2_body,
        grid_spec=grid_spec,
        out_shape=jax.ShapeDtypeStruct((_DISP, D_MODEL), jnp.float32),
        compiler_params=pltpu.CompilerParams(
            dimension_semantics=("arbitrary",),
        ),
    )(bexp, nused, h, w2, b2[:, None, :])


_A_PER_W = _NA // _NW
_G_CH = 32


def _sc_gather_body(eo_hbm, d0_hbm, d1_hbm, out_hbm, dv0, dv1, rows0, rows1,
                    sem):
    wid = lax.axis_index("s") * _NC + lax.axis_index("c")
    base_t = wid * _T_PER_W
    pltpu.sync_copy(d0_hbm.at[wid], dv0)
    pltpu.sync_copy(d1_hbm.at[wid], dv1)
    for j in range(_T_PER_W // _G_CH):
        cp0 = pltpu.async_copy(eo_hbm.at[dv0.at[j]], rows0, sem)
        cp1 = pltpu.async_copy(eo_hbm.at[dv1.at[j]], rows1, sem)
        cp0.wait()
        cp1.wait()
        tb = base_t + j * _G_CH
        pltpu.sync_copy(rows0, out_hbm.at[pl.ds(tb, _G_CH), 0, :])
        pltpu.sync_copy(rows1, out_hbm.at[pl.ds(tb, _G_CH), 1, :])


def _sc_gather(eo, dest0_3dg, dest1_3dg):
    mesh = plsc.VectorSubcoreMesh(core_axis_name="c", subcore_axis_name="s")
    nj = _T_PER_W // _G_CH
    return pl.kernel(
        _sc_gather_body,
        out_type=jax.ShapeDtypeStruct((N_TOKENS, 2, D_MODEL), jnp.float32),
        mesh=mesh,
        scratch_types=[
            pltpu.VMEM((nj, _G_CH), jnp.int32),
            pltpu.VMEM((nj, _G_CH), jnp.int32),
            pltpu.VMEM((_G_CH, D_MODEL), jnp.float32),
            pltpu.VMEM((_G_CH, D_MODEL), jnp.float32),
            pltpu.SemaphoreType.DMA,
        ],
    )(eo, dest0_3dg, dest1_3dg)


_BC = 512


def _combine_body(buf_ref, g_ref, y_ref):
    g = g_ref[...]
    cols = jax.lax.broadcasted_iota(jnp.int32, g.shape, 1)
    g0 = jnp.sum(jnp.where(cols == 0, g, 0.0), axis=1, keepdims=True)
    g1 = jnp.sum(jnp.where(cols == 1, g, 0.0), axis=1, keepdims=True)
    s = g0 * buf_ref[:, 0, :] + g1 * buf_ref[:, 1, :]
    y_ref[...] = jnp.log(jnp.where(s == 0.0, _EPS, s))


def _combine(buf, g01):
    return pl.pallas_call(
        _combine_body,
        grid=(N_TOKENS // _BC,),
        in_specs=[
            pl.BlockSpec((_BC, 2, D_MODEL), lambda t: (t, 0, 0)),
            pl.BlockSpec((_BC, 2), lambda t: (t, 0)),
        ],
        out_specs=pl.BlockSpec((_BC, D_MODEL), lambda t: (t, 0)),
        out_shape=jax.ShapeDtypeStruct((N_TOKENS, D_MODEL), jnp.float32),
    )(buf, g01)


def kernel(x, w_gate, w1, b1, w2, b2):
    g01, dest, bexp, nused, loss = _router(x, w_gate)
    nj = _T_PER_W // _SC_CH
    dest0_3d = jnp.reshape(dest[:, 0], (_NW, nj, _SC_CH))
    dest1_3d = jnp.reshape(dest[:, 1], (_NW, nj, _SC_CH))
    xd = _sc_scatter(x, dest0_3d, dest1_3d)
    bexp_v = jnp.reshape(bexp, (_NB,))
    nused_v = jnp.reshape(nused, (1,))
    h = _mlp1(xd, w1, b1, bexp_v, nused_v)
    eo = _mlp2(h, w2, b2, bexp_v, nused_v)
    njg = _T_PER_W // _G_CH
    dest0_3dg = jnp.reshape(dest[:, 0], (_NW, njg, _G_CH))
    dest1_3dg = jnp.reshape(dest[:, 1], (_NW, njg, _G_CH))
    buf = _sc_gather(eo, dest0_3dg, dest1_3dg)
    y = _combine(buf, g01)
    return (y, loss[0, 0])


# speedup vs baseline: 1.0651x; 1.0026x over previous
import functools

import jax
import jax.numpy as jnp
import numpy as np
from jax import lax
from jax.experimental import pallas as pl
from jax.experimental.pallas import tpu as pltpu
from jax.experimental.pallas import tpu_sc as plsc

NUM_EXPERTS = 8
TOP_K = 2
D_MODEL = 1024
D_FF = 4096
N_TOKENS = 4096

_BT = 256
_NB = (N_TOKENS * TOP_K + NUM_EXPERTS * (_BT - 1) + _BT - 1) // _BT
_DISP = _NB * _BT
_NA = N_TOKENS * TOP_K

_EPS = np.float32(np.finfo(float).eps)
_PREC = jax.lax.Precision.DEFAULT

_NC = 2
_NS = 16
_NW = _NC * _NS


def _dot(a, b):
    return jax.lax.dot_general(a, b, (((1,), (0,)), ((), ())),
                               preferred_element_type=jnp.float32,
                               precision=_PREC)


_CH = 512
_NCH = N_TOKENS // _CH


def _router_body(x_ref, wg_ref, g01_ref, dest_ref, bexp_ref, nused_ref,
                 loss_ref, m_ref, oh1_ref):
    x = x_ref[...]
    logits = _dot(x, wg_ref[...])
    e_ids = jax.lax.broadcasted_iota(jnp.int32, logits.shape, 1)

    m1 = jnp.max(logits, axis=1, keepdims=True)
    idx1 = jnp.min(jnp.where(logits == m1, e_ids, NUM_EXPERTS), axis=1,
                   keepdims=True)
    masked = jnp.where(e_ids == idx1, -jnp.inf, logits)
    m2 = jnp.max(masked, axis=1, keepdims=True)
    idx2 = jnp.min(jnp.where(masked == m2, e_ids, NUM_EXPERTS), axis=1,
                   keepdims=True)

    ex = jnp.exp(m2 - m1)
    denom = 1.0 + ex
    g1 = 1.0 / denom
    g2 = ex / denom
    g01_ref[...] = jnp.concatenate([g1, g2], axis=1)

    oh1 = (e_ids == idx1).astype(jnp.float32)
    oh2 = (e_ids == idx2).astype(jnp.float32)
    sel = oh1 + oh2
    m_ref[...] = sel
    oh1_ref[...] = oh1

    gates = oh1 * g1 + oh2 * g2
    importance = jnp.sum(gates, axis=0)
    load = jnp.sum((gates > 0.0).astype(jnp.float32), axis=0)

    def cv_sq(v):
        mean = jnp.mean(v)
        var = jnp.sum((v - mean) ** 2) / np.float32(NUM_EXPERTS - 1)
        return var / (mean * mean + np.float32(1e-10))

    loss = (cv_sq(importance) + cv_sq(load)) * np.float32(0.01)
    loss_ref[...] = jnp.reshape(loss, (1, 1))

    counts = jnp.sum(sel, axis=0, keepdims=True)
    nblk = jnp.ceil(counts / np.float32(_BT))
    r8 = jax.lax.broadcasted_iota(jnp.int32, (NUM_EXPERTS, NUM_EXPERTS), 0)
    c8 = jax.lax.broadcasted_iota(jnp.int32, (NUM_EXPERTS, NUM_EXPERTS), 1)
    upper8 = (r8 < c8).astype(jnp.float32)
    blk_off = _dot(nblk, upper8)
    off_rows = blk_off * np.float32(_BT)
    ends = jnp.reshape(blk_off + nblk, (NUM_EXPERTS, 1))
    used = jnp.sum(nblk)

    bb = jax.lax.broadcasted_iota(jnp.int32, (NUM_EXPERTS, _NB), 1)
    raw = jnp.sum((bb.astype(jnp.float32) >= ends).astype(jnp.float32),
                  axis=0, keepdims=True)
    eid8 = jax.lax.broadcasted_iota(jnp.int32, (1, NUM_EXPERTS), 1)
    laste = jnp.max(jnp.where(counts > 0.0, eid8, -1)).astype(jnp.float32)
    bexp_ref[...] = jnp.minimum(raw, laste).astype(jnp.int32)
    nused_ref[...] = jnp.reshape(used, (1, 1)).astype(jnp.int32)

    rr = jax.lax.broadcasted_iota(jnp.int32, (_CH, _CH), 0)
    cc = jax.lax.broadcasted_iota(jnp.int32, (_CH, _CH), 1)
    lower = (rr > cc).astype(jnp.float32)

    def chunk(c, run):
        mc = m_ref[pl.ds(c * _CH, _CH), :]
        oh1c = oh1_ref[pl.ds(c * _CH, _CH), :]
        oh2c = mc - oh1c
        pos = _dot(lower, mc) + run
        slot = pos + off_rows
        d0 = jnp.sum(oh1c * slot, axis=1, keepdims=True)
        d1 = jnp.sum(oh2c * slot, axis=1, keepdims=True)
        dest_ref[pl.ds(c * _CH, _CH), :] = jnp.concatenate(
            [d0, d1], axis=1).astype(jnp.int32)
        return run + jnp.sum(mc, axis=0, keepdims=True)

    jax.lax.fori_loop(0, _NCH, chunk, jnp.zeros((1, NUM_EXPERTS),
                                                jnp.float32))


def _router(x, w_gate):
    return pl.pallas_call(
        _router_body,
        out_shape=(
            jax.ShapeDtypeStruct((N_TOKENS, 2), jnp.float32),
            jax.ShapeDtypeStruct((N_TOKENS, 2), jnp.int32),
            jax.ShapeDtypeStruct((1, _NB), jnp.int32),
            jax.ShapeDtypeStruct((1, 1), jnp.int32),
            jax.ShapeDtypeStruct((1, 1), jnp.float32),
        ),
        scratch_shapes=[
            pltpu.VMEM((N_TOKENS, NUM_EXPERTS), jnp.float32),
            pltpu.VMEM((N_TOKENS, NUM_EXPERTS), jnp.float32),
        ],
    )(x, w_gate)


_T_PER_W = N_TOKENS // _NW
_SC_CH = 16


def _sc_scatter_body(x_hbm, d0_hbm, d1_hbm, xd_hbm, dv0, dv1, xv0, xv1,
                     rsem, wsem):
    wid = lax.axis_index("s") * _NC + lax.axis_index("c")
    base_t = wid * _T_PER_W
    pltpu.sync_copy(d0_hbm.at[wid], dv0)
    pltpu.sync_copy(d1_hbm.at[wid], dv1)
    nj = _T_PER_W // _SC_CH
    bufs = (xv0, xv1)

    reads = [pltpu.async_copy(x_hbm.at[pl.ds(base_t, _SC_CH)], bufs[0],
                              rsem)]
    writes = []
    for j in range(nj):
        reads[j].wait()
        if j >= 1:
            writes[2 * (j - 1)].wait()
            writes[2 * (j - 1) + 1].wait()
        if j + 1 < nj:
            reads.append(pltpu.async_copy(
                x_hbm.at[pl.ds(base_t + (j + 1) * _SC_CH, _SC_CH)],
                bufs[(j + 1) % 2], rsem))
        xv = bufs[j % 2]
        writes.append(pltpu.async_copy(xv, xd_hbm.at[dv0[j]], wsem))
        writes.append(pltpu.async_copy(xv, xd_hbm.at[dv1[j]], wsem))
    writes[-2].wait()
    writes[-1].wait()


def _sc_scatter(x, dest0_3d, dest1_3d):
    mesh = plsc.VectorSubcoreMesh(core_axis_name="c", subcore_axis_name="s")
    nj = _T_PER_W // _SC_CH
    return pl.kernel(
        _sc_scatter_body,
        out_type=jax.ShapeDtypeStruct((_DISP, D_MODEL), jnp.float32),
        mesh=mesh,
        scratch_types=[
            pltpu.VMEM((nj, _SC_CH), jnp.int32),
            pltpu.VMEM((nj, _SC_CH), jnp.int32),
            pltpu.VMEM((_SC_CH, D_MODEL), jnp.float32),
            pltpu.VMEM((_SC_CH, D_MODEL), jnp.float32),
            pltpu.SemaphoreType.DMA,
            pltpu.SemaphoreType.DMA,
        ],
    )(x, dest0_3d, dest1_3d)


def _mlp1_body(bexp_ref, nu_ref, xd_ref, w1_ref, b1_ref, h_ref):
    b = pl.program_id(0)

    @pl.when(b < nu_ref[0])
    def _():
        xb = xd_ref[...].astype(jnp.bfloat16)
        w1b = w1_ref[0].astype(jnp.bfloat16)
        h = jnp.maximum(_dot(xb, w1b) + b1_ref[0], 0.0)
        h_ref[...] = h.astype(jnp.bfloat16)


def _mlp1(xd, w1, b1, bexp, nused):
    grid_spec = pltpu.PrefetchScalarGridSpec(
        num_scalar_prefetch=2,
        grid=(_NB,),
        in_specs=[
            pl.BlockSpec((_BT, D_MODEL),
                         lambda b, be, nu: (jnp.minimum(b, nu[0] - 1), 0)),
            pl.BlockSpec((1, D_MODEL, D_FF), lambda b, be, nu: (be[b], 0, 0)),
            pl.BlockSpec((1, 1, D_FF), lambda b, be, nu: (be[b], 0, 0)),
        ],
        out_specs=pl.BlockSpec((_BT, D_FF), lambda b, be, nu: (b, 0)),
    )
    return pl.pallas_call(
        _mlp1_body,
        grid_spec=grid_spec,
        out_shape=jax.ShapeDtypeStruct((_DISP, D_FF), jnp.bfloat16),
        compiler_params=pltpu.CompilerParams(
            dimension_semantics=("arbitrary",),
        ),
    )(bexp, nused, xd, w1, b1[:, None, :])


def _mlp2_body(bexp_ref, nu_ref, h_ref, w2_ref, b2_ref, eo_ref):
    b = pl.program_id(0)

    @pl.when(b < nu_ref[0])
    def _():
        w2b = w2_ref[0].astype(jnp.bfloat16)
        eo_ref[...] = jnp.exp(_dot(h_ref[...], w2b) + b2_ref[0])


def _mlp2(h, w2, b2, bexp, nused):
    grid_spec = pltpu.PrefetchScalarGridSpec(
        num_scalar_prefetch=2,
        grid=(_NB,),
        in_specs=[
            pl.BlockSpec((_BT, D_FF),
                         lambda b, be, nu: (jnp.minimum(b, nu[0] - 1), 0)),
            pl.BlockSpec((1, D_FF, D_MODEL), lambda b, be, nu: (be[b], 0, 0)),
            pl.BlockSpec((1, 1, D_MODEL), lambda b, be, nu: (be[b], 0, 0)),
        ],
        out_specs=pl.BlockSpec((_BT, D_MODEL), lambda b, be, nu: (b, 0)),
    )
    return pl.pallas_call(
        _mlp2_body,
        grid_spec=grid_spec,
        out_shape=jax.ShapeDtypeStruct((_DISP, D_MODEL), jnp.float32),
        compiler_params=pltpu.CompilerParams(
            dimension_semantics=("arbitrary",),
        ),
    )(bexp, nused, h, w2, b2[:, None, :])


_A_PER_W = _NA // _NW
_G_CH = 16


def _sc_gather_body(eo_hbm, d0_hbm, d1_hbm, out_hbm, dv0, dv1,
                    ra0, ra1, rb0, rb1, rsem, wsem):
    wid = lax.axis_index("s") * _NC + lax.axis_index("c")
    base_t = wid * _T_PER_W
    pltpu.sync_copy(d0_hbm.at[wid], dv0)
    pltpu.sync_copy(d1_hbm.at[wid], dv1)
    nj = _T_PER_W // _G_CH
    pairs = ((ra0, ra1), (rb0, rb1))

    reads = [(pltpu.async_copy(eo_hbm.at[dv0.at[0]], ra0, rsem),
              pltpu.async_copy(eo_hbm.at[dv1.at[0]], ra1, rsem))]
    writes = []
    for j in range(nj):
        reads[j][0].wait()
        reads[j][1].wait()
        if j >= 1:
            writes[j - 1][0].wait()
            writes[j - 1][1].wait()
        if j + 1 < nj:
            nxt = pairs[(j + 1) % 2]
            reads.append(
                (pltpu.async_copy(eo_hbm.at[dv0.at[j + 1]], nxt[0], rsem),
                 pltpu.async_copy(eo_hbm.at[dv1.at[j + 1]], nxt[1], rsem)))
        r0, r1 = pairs[j % 2]
        tb = base_t + j * _G_CH
        writes.append(
            (pltpu.async_copy(r0, out_hbm.at[pl.ds(tb, _G_CH), 0, :], wsem),
             pltpu.async_copy(r1, out_hbm.at[pl.ds(tb, _G_CH), 1, :], wsem)))
    writes[-1][0].wait()
    writes[-1][1].wait()


def _sc_gather(eo, dest0_3dg, dest1_3dg):
    mesh = plsc.VectorSubcoreMesh(core_axis_name="c", subcore_axis_name="s")
    nj = _T_PER_W // _G_CH
    return pl.kernel(
        _sc_gather_body,
        out_type=jax.ShapeDtypeStruct((N_TOKENS, 2, D_MODEL), jnp.float32),
        mesh=mesh,
        scratch_types=[
            pltpu.VMEM((nj, _G_CH), jnp.int32),
            pltpu.VMEM((nj, _G_CH), jnp.int32),
            pltpu.VMEM((_G_CH, D_MODEL), jnp.float32),
            pltpu.VMEM((_G_CH, D_MODEL), jnp.float32),
            pltpu.VMEM((_G_CH, D_MODEL), jnp.float32),
            pltpu.VMEM((_G_CH, D_MODEL), jnp.float32),
            pltpu.SemaphoreType.DMA,
            pltpu.SemaphoreType.DMA,
        ],
    )(eo, dest0_3dg, dest1_3dg)


_BC = 512


def _combine_body(buf_ref, g_ref, y_ref):
    g = g_ref[...]
    cols = jax.lax.broadcasted_iota(jnp.int32, g.shape, 1)
    g0 = jnp.sum(jnp.where(cols == 0, g, 0.0), axis=1, keepdims=True)
    g1 = jnp.sum(jnp.where(cols == 1, g, 0.0), axis=1, keepdims=True)
    s = g0 * buf_ref[:, 0, :] + g1 * buf_ref[:, 1, :]
    y_ref[...] = jnp.log(jnp.where(s == 0.0, _EPS, s))


def _combine(buf, g01):
    return pl.pallas_call(
        _combine_body,
        grid=(N_TOKENS // _BC,),
        in_specs=[
            pl.BlockSpec((_BC, 2, D_MODEL), lambda t: (t, 0, 0)),
            pl.BlockSpec((_BC, 2), lambda t: (t, 0)),
        ],
        out_specs=pl.BlockSpec((_BC, D_MODEL), lambda t: (t, 0)),
        out_shape=jax.ShapeDtypeStruct((N_TOKENS, D_MODEL), jnp.float32),
    )(buf, g01)


def kernel(x, w_gate, w1, b1, w2, b2):
    g01, dest, bexp, nused, loss = _router(x, w_gate)
    nj = _T_PER_W // _SC_CH
    dest0_3d = jnp.reshape(dest[:, 0], (_NW, nj, _SC_CH))
    dest1_3d = jnp.reshape(dest[:, 1], (_NW, nj, _SC_CH))
    xd = _sc_scatter(x, dest0_3d, dest1_3d)
    bexp_v = jnp.reshape(bexp, (_NB,))
    nused_v = jnp.reshape(nused, (1,))
    h = _mlp1(xd, w1, b1, bexp_v, nused_v)
    eo = _mlp2(h, w2, b2, bexp_v, nused_v)
    njg = _T_PER_W // _G_CH
    dest0_3dg = jnp.reshape(dest[:, 0], (_NW, njg, _G_CH))
    dest1_3dg = jnp.reshape(dest[:, 1], (_NW, njg, _G_CH))
    buf = _sc_gather(eo, dest0_3dg, dest1_3dg)
    y = _combine(buf, g01)
    return (y, loss[0, 0])


# speedup vs baseline: 1.0749x; 1.0092x over previous
import functools

import jax
import jax.numpy as jnp
import numpy as np
from jax import lax
from jax.experimental import pallas as pl
from jax.experimental.pallas import tpu as pltpu
from jax.experimental.pallas import tpu_sc as plsc

NUM_EXPERTS = 8
TOP_K = 2
D_MODEL = 1024
D_FF = 4096
N_TOKENS = 4096

_BT = 256
_NB = (N_TOKENS * TOP_K + NUM_EXPERTS * (_BT - 1) + _BT - 1) // _BT
_DISP = _NB * _BT
_NA = N_TOKENS * TOP_K

_EPS = np.float32(np.finfo(float).eps)
_PREC = jax.lax.Precision.DEFAULT

_NC = 2
_NS = 16
_NW = _NC * _NS


def _dot(a, b):
    return jax.lax.dot_general(a, b, (((1,), (0,)), ((), ())),
                               preferred_element_type=jnp.float32,
                               precision=_PREC)


_CH = 512
_NCH = N_TOKENS // _CH


def _router_body(x_ref, wg_ref, g01_ref, dest_ref, bexp_ref, nused_ref,
                 loss_ref, m_ref, oh1_ref):
    x = x_ref[...]
    logits = _dot(x, wg_ref[...])
    e_ids = jax.lax.broadcasted_iota(jnp.int32, logits.shape, 1)

    m1 = jnp.max(logits, axis=1, keepdims=True)
    idx1 = jnp.min(jnp.where(logits == m1, e_ids, NUM_EXPERTS), axis=1,
                   keepdims=True)
    masked = jnp.where(e_ids == idx1, -jnp.inf, logits)
    m2 = jnp.max(masked, axis=1, keepdims=True)
    idx2 = jnp.min(jnp.where(masked == m2, e_ids, NUM_EXPERTS), axis=1,
                   keepdims=True)

    ex = jnp.exp(m2 - m1)
    denom = 1.0 + ex
    g1 = 1.0 / denom
    g2 = ex / denom
    g01_ref[...] = jnp.concatenate([g1, g2], axis=1)

    oh1 = (e_ids == idx1).astype(jnp.float32)
    oh2 = (e_ids == idx2).astype(jnp.float32)
    sel = oh1 + oh2
    m_ref[...] = sel
    oh1_ref[...] = oh1

    gates = oh1 * g1 + oh2 * g2
    importance = jnp.sum(gates, axis=0)
    load = jnp.sum((gates > 0.0).astype(jnp.float32), axis=0)

    def cv_sq(v):
        mean = jnp.mean(v)
        var = jnp.sum((v - mean) ** 2) / np.float32(NUM_EXPERTS - 1)
        return var / (mean * mean + np.float32(1e-10))

    loss = (cv_sq(importance) + cv_sq(load)) * np.float32(0.01)
    loss_ref[...] = jnp.reshape(loss, (1, 1))

    counts = jnp.sum(sel, axis=0, keepdims=True)
    nblk = jnp.ceil(counts / np.float32(_BT))
    r8 = jax.lax.broadcasted_iota(jnp.int32, (NUM_EXPERTS, NUM_EXPERTS), 0)
    c8 = jax.lax.broadcasted_iota(jnp.int32, (NUM_EXPERTS, NUM_EXPERTS), 1)
    upper8 = (r8 < c8).astype(jnp.float32)
    blk_off = _dot(nblk, upper8)
    off_rows = blk_off * np.float32(_BT)
    ends = jnp.reshape(blk_off + nblk, (NUM_EXPERTS, 1))
    used = jnp.sum(nblk)

    bb = jax.lax.broadcasted_iota(jnp.int32, (NUM_EXPERTS, _NB), 1)
    raw = jnp.sum((bb.astype(jnp.float32) >= ends).astype(jnp.float32),
                  axis=0, keepdims=True)
    eid8 = jax.lax.broadcasted_iota(jnp.int32, (1, NUM_EXPERTS), 1)
    laste = jnp.max(jnp.where(counts > 0.0, eid8, -1)).astype(jnp.float32)
    bexp_ref[...] = jnp.minimum(raw, laste).astype(jnp.int32)
    nused_ref[...] = jnp.reshape(used, (1, 1)).astype(jnp.int32)

    rr = jax.lax.broadcasted_iota(jnp.int32, (_CH, _CH), 0)
    cc = jax.lax.broadcasted_iota(jnp.int32, (_CH, _CH), 1)
    lower = (rr > cc).astype(jnp.float32)

    def chunk(c, run):
        mc = m_ref[pl.ds(c * _CH, _CH), :]
        oh1c = oh1_ref[pl.ds(c * _CH, _CH), :]
        oh2c = mc - oh1c
        pos = _dot(lower, mc) + run
        slot = pos + off_rows
        d0 = jnp.sum(oh1c * slot, axis=1, keepdims=True)
        d1 = jnp.sum(oh2c * slot, axis=1, keepdims=True)
        dest_ref[pl.ds(c * _CH, _CH), :] = jnp.concatenate(
            [d0, d1], axis=1).astype(jnp.int32)
        return run + jnp.sum(mc, axis=0, keepdims=True)

    jax.lax.fori_loop(0, _NCH, chunk, jnp.zeros((1, NUM_EXPERTS),
                                                jnp.float32))


def _router(x, w_gate):
    return pl.pallas_call(
        _router_body,
        out_shape=(
            jax.ShapeDtypeStruct((N_TOKENS, 2), jnp.float32),
            jax.ShapeDtypeStruct((N_TOKENS, 2), jnp.int32),
            jax.ShapeDtypeStruct((1, _NB), jnp.int32),
            jax.ShapeDtypeStruct((1, 1), jnp.int32),
            jax.ShapeDtypeStruct((1, 1), jnp.float32),
        ),
        scratch_shapes=[
            pltpu.VMEM((N_TOKENS, NUM_EXPERTS), jnp.float32),
            pltpu.VMEM((N_TOKENS, NUM_EXPERTS), jnp.float32),
        ],
    )(x, w_gate)


_T_PER_W = N_TOKENS // _NW
_SC_CH = 16


def _sc_scatter_body(x_hbm, d0_hbm, d1_hbm, xd_hbm, dv0, dv1, xv0, xv1,
                     rsem, wsem):
    wid = lax.axis_index("s") * _NC + lax.axis_index("c")
    base_t = wid * _T_PER_W
    pltpu.sync_copy(d0_hbm.at[wid], dv0)
    pltpu.sync_copy(d1_hbm.at[wid], dv1)
    nj = _T_PER_W // _SC_CH
    bufs = (xv0, xv1)

    reads = [pltpu.async_copy(x_hbm.at[pl.ds(base_t, _SC_CH)], bufs[0],
                              rsem)]
    writes = []
    for j in range(nj):
        reads[j].wait()
        if j >= 1:
            writes[2 * (j - 1)].wait()
            writes[2 * (j - 1) + 1].wait()
        if j + 1 < nj:
            reads.append(pltpu.async_copy(
                x_hbm.at[pl.ds(base_t + (j + 1) * _SC_CH, _SC_CH)],
                bufs[(j + 1) % 2], rsem))
        xv = bufs[j % 2]
        writes.append(pltpu.async_copy(xv, xd_hbm.at[dv0[j]], wsem))
        writes.append(pltpu.async_copy(xv, xd_hbm.at[dv1[j]], wsem))
    writes[-2].wait()
    writes[-1].wait()


def _sc_scatter(x, dest0_3d, dest1_3d):
    mesh = plsc.VectorSubcoreMesh(core_axis_name="c", subcore_axis_name="s")
    nj = _T_PER_W // _SC_CH
    return pl.kernel(
        _sc_scatter_body,
        out_type=jax.ShapeDtypeStruct((_DISP, D_MODEL), jnp.float32),
        mesh=mesh,
        scratch_types=[
            pltpu.VMEM((nj, _SC_CH), jnp.int32),
            pltpu.VMEM((nj, _SC_CH), jnp.int32),
            pltpu.VMEM((_SC_CH, D_MODEL), jnp.float32),
            pltpu.VMEM((_SC_CH, D_MODEL), jnp.float32),
            pltpu.SemaphoreType.DMA,
            pltpu.SemaphoreType.DMA,
        ],
    )(x, dest0_3d, dest1_3d)


def _mlp1_body(bexp_ref, nu_ref, xd_ref, w1_ref, b1_ref, h_ref):
    b = pl.program_id(0)

    @pl.when(b < nu_ref[0])
    def _():
        xb = xd_ref[...].astype(jnp.bfloat16)
        w1b = w1_ref[0].astype(jnp.bfloat16)
        h = jnp.maximum(_dot(xb, w1b) + b1_ref[0], 0.0)
        h_ref[...] = h.astype(jnp.bfloat16)


def _mlp1(xd, w1, b1, bexp, nused):
    grid_spec = pltpu.PrefetchScalarGridSpec(
        num_scalar_prefetch=2,
        grid=(_NB,),
        in_specs=[
            pl.BlockSpec((_BT, D_MODEL),
                         lambda b, be, nu: (jnp.minimum(b, nu[0] - 1), 0)),
            pl.BlockSpec((1, D_MODEL, D_FF), lambda b, be, nu: (be[b], 0, 0)),
            pl.BlockSpec((1, 1, D_FF), lambda b, be, nu: (be[b], 0, 0)),
        ],
        out_specs=pl.BlockSpec((_BT, D_FF), lambda b, be, nu: (b, 0)),
    )
    return pl.pallas_call(
        _mlp1_body,
        grid_spec=grid_spec,
        out_shape=jax.ShapeDtypeStruct((_DISP, D_FF), jnp.bfloat16),
        compiler_params=pltpu.CompilerParams(
            dimension_semantics=("arbitrary",),
        ),
    )(bexp, nused, xd, w1, b1[:, None, :])


def _mlp2_body(bexp_ref, nu_ref, h_ref, w2_ref, b2_ref, eo_ref):
    b = pl.program_id(0)

    @pl.when(b < nu_ref[0])
    def _():
        w2b = w2_ref[0].astype(jnp.bfloat16)
        eo_ref[...] = jnp.exp(_dot(h_ref[...], w2b) + b2_ref[0])


def _mlp2(h, w2, b2, bexp, nused):
    grid_spec = pltpu.PrefetchScalarGridSpec(
        num_scalar_prefetch=2,
        grid=(_NB,),
        in_specs=[
            pl.BlockSpec((_BT, D_FF),
                         lambda b, be, nu: (jnp.minimum(b, nu[0] - 1), 0)),
            pl.BlockSpec((1, D_FF, D_MODEL), lambda b, be, nu: (be[b], 0, 0)),
            pl.BlockSpec((1, 1, D_MODEL), lambda b, be, nu: (be[b], 0, 0)),
        ],
        out_specs=pl.BlockSpec((_BT, D_MODEL), lambda b, be, nu: (b, 0)),
    )
    return pl.pallas_call(
        _mlp2_body,
        grid_spec=grid_spec,
        out_shape=jax.ShapeDtypeStruct((_DISP, D_MODEL), jnp.float32),
        compiler_params=pltpu.CompilerParams(
            dimension_semantics=("arbitrary",),
        ),
    )(bexp, nused, h, w2, b2[:, None, :])


_A_PER_W = _NA // _NW
_G_CH = 16


def _sc_gather_body(eo_hbm, d0_hbm, d1_hbm, out_hbm, dv0, dv1,
                    ra0, ra1, rb0, rb1, rsem, wsem):
    wid = lax.axis_index("s") * _NC + lax.axis_index("c")
    base_t = wid * _T_PER_W
    pltpu.sync_copy(d0_hbm.at[wid], dv0)
    pltpu.sync_copy(d1_hbm.at[wid], dv1)
    nj = _T_PER_W // _G_CH
    pairs = ((ra0, ra1), (rb0, rb1))

    reads = [(pltpu.async_copy(eo_hbm.at[dv0.at[0]], ra0, rsem),
              pltpu.async_copy(eo_hbm.at[dv1.at[0]], ra1, rsem))]
    writes = []
    for j in range(nj):
        reads[j][0].wait()
        reads[j][1].wait()
        if j >= 1:
            writes[j - 1][0].wait()
            writes[j - 1][1].wait()
        if j + 1 < nj:
            nxt = pairs[(j + 1) % 2]
            reads.append(
                (pltpu.async_copy(eo_hbm.at[dv0.at[j + 1]], nxt[0], rsem),
                 pltpu.async_copy(eo_hbm.at[dv1.at[j + 1]], nxt[1], rsem)))
        r0, r1 = pairs[j % 2]
        tb = base_t + j * _G_CH
        writes.append(
            (pltpu.async_copy(r0, out_hbm.at[0, pl.ds(tb, _G_CH), :], wsem),
             pltpu.async_copy(r1, out_hbm.at[1, pl.ds(tb, _G_CH), :], wsem)))
    writes[-1][0].wait()
    writes[-1][1].wait()


def _sc_gather(eo, dest0_3dg, dest1_3dg):
    mesh = plsc.VectorSubcoreMesh(core_axis_name="c", subcore_axis_name="s")
    nj = _T_PER_W // _G_CH
    return pl.kernel(
        _sc_gather_body,
        out_type=jax.ShapeDtypeStruct((2, N_TOKENS, D_MODEL), jnp.float32),
        mesh=mesh,
        scratch_types=[
            pltpu.VMEM((nj, _G_CH), jnp.int32),
            pltpu.VMEM((nj, _G_CH), jnp.int32),
            pltpu.VMEM((_G_CH, D_MODEL), jnp.float32),
            pltpu.VMEM((_G_CH, D_MODEL), jnp.float32),
            pltpu.VMEM((_G_CH, D_MODEL), jnp.float32),
            pltpu.VMEM((_G_CH, D_MODEL), jnp.float32),
            pltpu.SemaphoreType.DMA,
            pltpu.SemaphoreType.DMA,
        ],
    )(eo, dest0_3dg, dest1_3dg)


_BC = 512


def _combine_body(buf_ref, g_ref, y_ref):
    g = g_ref[...]
    cols = jax.lax.broadcasted_iota(jnp.int32, g.shape, 1)
    g0 = jnp.sum(jnp.where(cols == 0, g, 0.0), axis=1, keepdims=True)
    g1 = jnp.sum(jnp.where(cols == 1, g, 0.0), axis=1, keepdims=True)
    s = g0 * buf_ref[0] + g1 * buf_ref[1]
    y_ref[...] = jnp.log(jnp.where(s == 0.0, _EPS, s))


def _combine(buf, g01):
    return pl.pallas_call(
        _combine_body,
        grid=(N_TOKENS // _BC,),
        in_specs=[
            pl.BlockSpec((2, _BC, D_MODEL), lambda t: (0, t, 0)),
            pl.BlockSpec((_BC, 2), lambda t: (t, 0)),
        ],
        out_specs=pl.BlockSpec((_BC, D_MODEL), lambda t: (t, 0)),
        out_shape=jax.ShapeDtypeStruct((N_TOKENS, D_MODEL), jnp.float32),
    )(buf, g01)


def kernel(x, w_gate, w1, b1, w2, b2):
    g01, dest, bexp, nused, loss = _router(x, w_gate)
    nj = _T_PER_W // _SC_CH
    dest0_3d = jnp.reshape(dest[:, 0], (_NW, nj, _SC_CH))
    dest1_3d = jnp.reshape(dest[:, 1], (_NW, nj, _SC_CH))
    xd = _sc_scatter(x, dest0_3d, dest1_3d)
    bexp_v = jnp.reshape(bexp, (_NB,))
    nused_v = jnp.reshape(nused, (1,))
    h = _mlp1(xd, w1, b1, bexp_v, nused_v)
    eo = _mlp2(h, w2, b2, bexp_v, nused_v)
    njg = _T_PER_W // _G_CH
    dest0_3dg = jnp.reshape(dest[:, 0], (_NW, njg, _G_CH))
    dest1_3dg = jnp.reshape(dest[:, 1], (_NW, njg, _G_CH))
    buf = _sc_gather(eo, dest0_3dg, dest1_3dg)
    y = _combine(buf, g01)
    return (y, loss[0, 0])
